# baseline (speedup 1.0000x reference)
import functools

import jax
import jax.numpy as jnp
from jax import lax
from jax.experimental import pallas as pl
from jax.experimental.pallas import tpu as pltpu
from jax.experimental.pallas import tpu_sc as plsc


def _s1_body(x_ref, w1_ref, o_ref):
    o_ref[...] = jnp.dot(x_ref[...], w1_ref[...],
                         preferred_element_type=jnp.float32)


def _layer1_body(adj_ref, s1_ref, b1_ref, w2_ref, t_ref):
    h = jnp.dot(adj_ref[...], s1_ref[...], preferred_element_type=jnp.float32)
    h = jnp.maximum(h + b1_ref[...], 0.0)
    t_ref[...] = jnp.dot(h, w2_ref[...], preferred_element_type=jnp.float32)


def _layer2_body(adj_ref, t_ref, b2_ref, dm_ref, e_ref):
    e = jnp.dot(adj_ref[...], t_ref[...], preferred_element_type=jnp.float32)
    e_ref[...] = (e + b2_ref[...]) * jnp.sqrt(dm_ref[...])


def _pick_bm(n):
    for bm in (256, 200, 128, 80, 40, 16, 8):
        if n % bm == 0:
            return bm
    return n


def _gcn_embeds(x, adj, w1, b1, w2, b2, distmult):
    n, feat = x.shape
    hid = w1.shape[1]
    out = w2.shape[1]
    bm = _pick_bm(n)

    s1 = pl.pallas_call(
        _s1_body,
        out_shape=jax.ShapeDtypeStruct((n, hid), jnp.float32),
        grid=(n // bm,),
        in_specs=[pl.BlockSpec((bm, feat), lambda i: (i, 0)),
                  pl.BlockSpec((feat, hid), lambda i: (0, 0))],
        out_specs=pl.BlockSpec((bm, hid), lambda i: (i, 0)),
    )(x, w1)

    t = pl.pallas_call(
        _layer1_body,
        out_shape=jax.ShapeDtypeStruct((n, out), jnp.float32),
        grid=(n // bm,),
        in_specs=[pl.BlockSpec((bm, n), lambda i: (i, 0)),
                  pl.BlockSpec((n, hid), lambda i: (0, 0)),
                  pl.BlockSpec((hid,), lambda i: (0,)),
                  pl.BlockSpec((hid, out), lambda i: (0, 0))],
        out_specs=pl.BlockSpec((bm, out), lambda i: (i, 0)),
    )(adj, s1, b1, w2)

    e = pl.pallas_call(
        _layer2_body,
        out_shape=jax.ShapeDtypeStruct((n, out), jnp.float32),
        grid=(n // bm,),
        in_specs=[pl.BlockSpec((bm, n), lambda i: (i, 0)),
                  pl.BlockSpec((n, out), lambda i: (0, 0)),
                  pl.BlockSpec((out,), lambda i: (0,)),
                  pl.BlockSpec((out,), lambda i: (0,))],
        out_specs=pl.BlockSpec((bm, out), lambda i: (i, 0)),
    )(adj, t, b2, distmult)

    return e


_CHUNK = 192


def _lane_take(x, idx):
    dnums = lax.GatherDimensionNumbers(
        offset_dims=(), collapsed_slice_dims=(0,), start_index_map=(0,))
    return lax.gather(x, idx[:, None], dnums, slice_sizes=(1,),
                      mode=lax.GatherScatterMode.PROMISE_IN_BOUNDS)


@functools.lru_cache(maxsize=None)
def _make_score_kernel(n, out, p_pad):
    info = plsc.get_sparse_core_info()
    nw = info.num_cores * info.num_subcores
    per_w = p_pad // nw
    chunk = _CHUNK
    n_chunks = per_w // chunk
    nlan = info.num_lanes

    mesh = plsc.VectorSubcoreMesh(core_axis_name="c", subcore_axis_name="s")

    @functools.partial(
        pl.kernel,
        out_type=jax.ShapeDtypeStruct((p_pad,), jnp.float32),
        mesh=mesh,
        scratch_types=[
            [pltpu.VMEM((chunk,), jnp.int32)] * 2,
            [pltpu.VMEM((chunk,), jnp.int32)] * 2,
            [pltpu.VMEM((chunk, out), jnp.float32)] * 2,
            [pltpu.VMEM((chunk, out), jnp.float32)] * 2,
            pltpu.VMEM((chunk,), jnp.float32),
            [pltpu.SemaphoreType.DMA] * 2,
            [pltpu.SemaphoreType.DMA] * 2,
        ],
    )
    def score(table, src, dst, o_hbm, sidx, didx, srows, drows, ovec,
              sem_s, sem_d):
        wid = lax.axis_index("s") * info.num_cores + lax.axis_index("c")
        base_w = wid * per_w
        lanes = lax.iota(jnp.int32, nlan)
        perms = [lanes ^ sh for sh in (8, 4, 2, 1)]

        pending = {}

        def prefetch(ci, slot):
            base = base_w + ci * chunk
            pltpu.sync_copy(src.at[pl.ds(base, chunk)], sidx[slot])
            pltpu.sync_copy(dst.at[pl.ds(base, chunk)], didx[slot])
            cps = pltpu.async_copy(table.at[sidx[slot]], srows[slot],
                                   sem_s[slot])
            cpd = pltpu.async_copy(table.at[didx[slot]], drows[slot],
                                   sem_d[slot])
            pending[slot] = (cps, cpd)

        prefetch(0, 0)
        for ci in range(n_chunks):
            slot = ci % 2
            base = base_w + ci * chunk
            cps, cpd = pending[slot]
            cps.wait()
            cpd.wait()
            if ci + 1 < n_chunks:
                prefetch(ci + 1, 1 - slot)
            sr = srows[slot]
            dr = drows[slot]

            def group(g, carry):
                def pair2(i2, res):
                    accs = []
                    for u in range(2):
                        pr = g * nlan + i2 * 2 + u
                        acc = sr[pr, pl.ds(0, nlan)] * dr[pr, pl.ds(0, nlan)]
                        for j in range(1, out // nlan):
                            acc = acc + (sr[pr, pl.ds(j * nlan, nlan)]
                                         * dr[pr, pl.ds(j * nlan, nlan)])
                        for perm in perms:
                            acc = acc + _lane_take(acc, perm)
                        accs.append(acc)
                    res = jnp.where(lanes == i2 * 2, accs[0], res)
                    return jnp.where(lanes == i2 * 2 + 1, accs[1], res)

                res = lax.fori_loop(0, nlan // 2, pair2,
                                    jnp.zeros((nlan,), jnp.float32))
                ovec[pl.ds(g * nlan, nlan)] = res
                return carry

            lax.fori_loop(0, chunk // nlan, group, 0)
            pltpu.sync_copy(ovec, o_hbm.at[pl.ds(base, chunk)])

    return score


def kernel(x, adj, to_pred, W1, b1, W2, b2, distmult):
    n = x.shape[0]
    out = W2.shape[1]
    p = to_pred.shape[0]

    e = _gcn_embeds(x, adj, W1, b1, W2, b2, distmult)

    info = plsc.get_sparse_core_info()
    nw = info.num_cores * info.num_subcores
    unit = nw * _CHUNK
    p_pad = ((p + unit - 1) // unit) * unit

    src = to_pred[:, 0]
    dst = to_pred[:, 1]
    if p_pad != p:
        zpad = jnp.zeros((p_pad - p,), jnp.int32)
        src = jnp.concatenate([src, zpad])
        dst = jnp.concatenate([dst, zpad])

    scores = _make_score_kernel(n, out, p_pad)(e, src, dst)
    return scores[:p]


# speedup vs baseline: 1.0058x; 1.0058x over previous
import functools

import jax
import jax.numpy as jnp
from jax import lax
from jax.experimental import pallas as pl
from jax.experimental.pallas import tpu as pltpu
from jax.experimental.pallas import tpu_sc as plsc


def _s1_body(x_ref, w1_ref, o_ref):
    o_ref[...] = jnp.dot(x_ref[...], w1_ref[...],
                         preferred_element_type=jnp.float32)


def _layer1_body(adj_ref, s1_ref, b1_ref, w2_ref, t_ref):
    h = jnp.dot(adj_ref[...], s1_ref[...], preferred_element_type=jnp.float32)
    h = jnp.maximum(h + b1_ref[...], 0.0)
    t_ref[...] = jnp.dot(h, w2_ref[...], preferred_element_type=jnp.float32)


def _layer2_body(adj_ref, t_ref, b2_ref, dm_ref, e_ref):
    e = jnp.dot(adj_ref[...], t_ref[...], preferred_element_type=jnp.float32)
    e_ref[...] = (e + b2_ref[...]) * jnp.sqrt(dm_ref[...])


def _pick_bm(n):
    for bm in (256, 200, 128, 80, 40, 16, 8):
        if n % bm == 0:
            return bm
    return n


def _gcn_embeds(x, adj, w1, b1, w2, b2, distmult):
    n, feat = x.shape
    hid = w1.shape[1]
    out = w2.shape[1]
    bm = _pick_bm(n)

    s1 = pl.pallas_call(
        _s1_body,
        out_shape=jax.ShapeDtypeStruct((n, hid), jnp.float32),
        grid=(n // bm,),
        in_specs=[pl.BlockSpec((bm, feat), lambda i: (i, 0)),
                  pl.BlockSpec((feat, hid), lambda i: (0, 0))],
        out_specs=pl.BlockSpec((bm, hid), lambda i: (i, 0)),
    )(x, w1)

    t = pl.pallas_call(
        _layer1_body,
        out_shape=jax.ShapeDtypeStruct((n, out), jnp.float32),
        grid=(n // bm,),
        in_specs=[pl.BlockSpec((bm, n), lambda i: (i, 0)),
                  pl.BlockSpec((n, hid), lambda i: (0, 0)),
                  pl.BlockSpec((hid,), lambda i: (0,)),
                  pl.BlockSpec((hid, out), lambda i: (0, 0))],
        out_specs=pl.BlockSpec((bm, out), lambda i: (i, 0)),
    )(adj, s1, b1, w2)

    e = pl.pallas_call(
        _layer2_body,
        out_shape=jax.ShapeDtypeStruct((n, out), jnp.float32),
        grid=(n // bm,),
        in_specs=[pl.BlockSpec((bm, n), lambda i: (i, 0)),
                  pl.BlockSpec((n, out), lambda i: (0, 0)),
                  pl.BlockSpec((out,), lambda i: (0,)),
                  pl.BlockSpec((out,), lambda i: (0,))],
        out_specs=pl.BlockSpec((bm, out), lambda i: (i, 0)),
    )(adj, t, b2, distmult)

    return e


_CHUNK = 192


def _lane_take(x, idx):
    dnums = lax.GatherDimensionNumbers(
        offset_dims=(), collapsed_slice_dims=(0,), start_index_map=(0,))
    return lax.gather(x, idx[:, None], dnums, slice_sizes=(1,),
                      mode=lax.GatherScatterMode.PROMISE_IN_BOUNDS)


@functools.lru_cache(maxsize=None)
def _make_score_kernel(n, out, p_pad):
    info = plsc.get_sparse_core_info()
    nw = info.num_cores * info.num_subcores
    per_w = p_pad // nw
    chunk = _CHUNK
    n_chunks = per_w // chunk
    nlan = info.num_lanes

    mesh = plsc.VectorSubcoreMesh(core_axis_name="c", subcore_axis_name="s")

    @functools.partial(
        pl.kernel,
        out_type=jax.ShapeDtypeStruct((p_pad,), jnp.float32),
        mesh=mesh,
        scratch_types=[
            [pltpu.VMEM((chunk,), jnp.int32)] * 2,
            [pltpu.VMEM((chunk,), jnp.int32)] * 2,
            [pltpu.VMEM((chunk, out), jnp.float32)] * 2,
            [pltpu.VMEM((chunk, out), jnp.float32)] * 2,
            pltpu.VMEM((chunk,), jnp.float32),
            [pltpu.SemaphoreType.DMA] * 2,
            [pltpu.SemaphoreType.DMA] * 2,
        ],
    )
    def score(table, src, dst, o_hbm, sidx, didx, srows, drows, ovec,
              sem_s, sem_d):
        wid = lax.axis_index("s") * info.num_cores + lax.axis_index("c")
        base_w = wid * per_w
        lanes = lax.iota(jnp.int32, nlan)
        perms = [lanes ^ sh for sh in (8, 4, 2, 1)]

        pending = {}

        def prefetch(ci, slot):
            base = base_w + ci * chunk
            pltpu.sync_copy(src.at[pl.ds(base, chunk)], sidx[slot])
            pltpu.sync_copy(dst.at[pl.ds(base, chunk)], didx[slot])
            cps = pltpu.async_copy(table.at[sidx[slot]], srows[slot],
                                   sem_s[slot])
            cpd = pltpu.async_copy(table.at[didx[slot]], drows[slot],
                                   sem_d[slot])
            pending[slot] = (cps, cpd)

        prefetch(0, 0)
        for ci in range(n_chunks):
            slot = ci % 2
            base = base_w + ci * chunk
            cps, cpd = pending[slot]
            cps.wait()
            cpd.wait()
            if ci + 1 < n_chunks:
                prefetch(ci + 1, 1 - slot)
            sr = srows[slot]
            dr = drows[slot]

            def group(g, carry):
                def pair(i, res):
                    pr = g * nlan + i
                    acc = sr[pr, pl.ds(0, nlan)] * dr[pr, pl.ds(0, nlan)]
                    for j in range(1, out // nlan):
                        acc = acc + (sr[pr, pl.ds(j * nlan, nlan)]
                                     * dr[pr, pl.ds(j * nlan, nlan)])
                    for perm in perms:
                        acc = acc + _lane_take(acc, perm)
                    return jnp.where(lanes == i, acc, res)

                res = lax.fori_loop(0, nlan, pair,
                                    jnp.zeros((nlan,), jnp.float32))
                ovec[pl.ds(g * nlan, nlan)] = res
                return carry

            lax.fori_loop(0, chunk // nlan, group, 0)
            pltpu.sync_copy(ovec, o_hbm.at[pl.ds(base, chunk)])

    return score


def kernel(x, adj, to_pred, W1, b1, W2, b2, distmult):
    n = x.shape[0]
    out = W2.shape[1]
    p = to_pred.shape[0]

    e = _gcn_embeds(x, adj, W1, b1, W2, b2, distmult)

    info = plsc.get_sparse_core_info()
    nw = info.num_cores * info.num_subcores
    unit = nw * _CHUNK
    p_pad = ((p + unit - 1) // unit) * unit

    src = to_pred[:, 0]
    dst = to_pred[:, 1]
    if p_pad != p:
        zpad = jnp.zeros((p_pad - p,), jnp.int32)
        src = jnp.concatenate([src, zpad])
        dst = jnp.concatenate([dst, zpad])

    scores = _make_score_kernel(n, out, p_pad)(e, src, dst)
    return scores[:p]


# speedup vs baseline: 1.2151x; 1.2081x over previous
import functools

import jax
import jax.numpy as jnp
from jax import lax
from jax.experimental import pallas as pl
from jax.experimental.pallas import tpu as pltpu
from jax.experimental.pallas import tpu_sc as plsc


def _s1_body(x_ref, w1_ref, o_ref):
    o_ref[...] = jnp.dot(x_ref[...], w1_ref[...],
                         preferred_element_type=jnp.float32)


def _layer1_body(adj_ref, s1_ref, b1_ref, w2_ref, t_ref):
    h = jnp.dot(adj_ref[...], s1_ref[...], preferred_element_type=jnp.float32)
    h = jnp.maximum(h + b1_ref[...], 0.0)
    t_ref[...] = jnp.dot(h, w2_ref[...], preferred_element_type=jnp.float32)


def _layer2_body(adj_ref, t_ref, b2_ref, dm_ref, e_ref):
    e = jnp.dot(adj_ref[...], t_ref[...], preferred_element_type=jnp.float32)
    e_ref[...] = (e + b2_ref[...]) * jnp.sqrt(dm_ref[...])


def _pick_bm(n):
    for bm in (256, 200, 128, 80, 40, 16, 8):
        if n % bm == 0:
            return bm
    return n


def _gcn_embeds(x, adj, w1, b1, w2, b2, distmult):
    n, feat = x.shape
    hid = w1.shape[1]
    out = w2.shape[1]
    bm = _pick_bm(n)

    s1 = pl.pallas_call(
        _s1_body,
        out_shape=jax.ShapeDtypeStruct((n, hid), jnp.float32),
        grid=(n // bm,),
        in_specs=[pl.BlockSpec((bm, feat), lambda i: (i, 0)),
                  pl.BlockSpec((feat, hid), lambda i: (0, 0))],
        out_specs=pl.BlockSpec((bm, hid), lambda i: (i, 0)),
    )(x, w1)

    t = pl.pallas_call(
        _layer1_body,
        out_shape=jax.ShapeDtypeStruct((n, out), jnp.float32),
        grid=(n // bm,),
        in_specs=[pl.BlockSpec((bm, n), lambda i: (i, 0)),
                  pl.BlockSpec((n, hid), lambda i: (0, 0)),
                  pl.BlockSpec((hid,), lambda i: (0,)),
                  pl.BlockSpec((hid, out), lambda i: (0, 0))],
        out_specs=pl.BlockSpec((bm, out), lambda i: (i, 0)),
    )(adj, s1, b1, w2)

    e = pl.pallas_call(
        _layer2_body,
        out_shape=jax.ShapeDtypeStruct((n, out), jnp.float32),
        grid=(n // bm,),
        in_specs=[pl.BlockSpec((bm, n), lambda i: (i, 0)),
                  pl.BlockSpec((n, out), lambda i: (0, 0)),
                  pl.BlockSpec((out,), lambda i: (0,)),
                  pl.BlockSpec((out,), lambda i: (0,))],
        out_specs=pl.BlockSpec((bm, out), lambda i: (i, 0)),
    )(adj, t, b2, distmult)

    return e


_CHUNK = 400


def _lane_take(x, idx):
    dnums = lax.GatherDimensionNumbers(
        offset_dims=(), collapsed_slice_dims=(0,), start_index_map=(0,))
    return lax.gather(x, idx[:, None], dnums, slice_sizes=(1,),
                      mode=lax.GatherScatterMode.PROMISE_IN_BOUNDS)


@functools.lru_cache(maxsize=None)
def _make_score_kernel(n, out, p_pad):
    info = plsc.get_sparse_core_info()
    nw = info.num_cores * info.num_subcores
    per_w = p_pad // nw
    chunk = _CHUNK
    n_chunks = per_w // chunk
    nlan = info.num_lanes

    mesh = plsc.VectorSubcoreMesh(core_axis_name="c", subcore_axis_name="s")

    @functools.partial(
        pl.kernel,
        out_type=jax.ShapeDtypeStruct((p_pad,), jnp.float32),
        mesh=mesh,
        scratch_types=[
            pltpu.VMEM((chunk,), jnp.int32),
            pltpu.VMEM((chunk,), jnp.int32),
            pltpu.VMEM((chunk, out), jnp.float32),
            pltpu.VMEM((chunk, out), jnp.float32),
            pltpu.VMEM((chunk,), jnp.float32),
            pltpu.SemaphoreType.DMA,
            pltpu.SemaphoreType.DMA,
        ],
    )
    def score(table, src, dst, o_hbm, sidx, didx, srows, drows, ovec,
              sem_s, sem_d):
        wid = lax.axis_index("s") * info.num_cores + lax.axis_index("c")
        base_w = wid * per_w
        lanes = lax.iota(jnp.int32, nlan)
        perms = [lanes ^ sh for sh in (8, 4, 2, 1)]
        for ci in range(n_chunks):
            base = base_w + ci * chunk
            pltpu.sync_copy(src.at[pl.ds(base, chunk)], sidx)
            pltpu.sync_copy(dst.at[pl.ds(base, chunk)], didx)
            cps = pltpu.async_copy(table.at[sidx], srows, sem_s)
            cpd = pltpu.async_copy(table.at[didx], drows, sem_d)
            cps.wait()
            cpd.wait()

            def group(g, carry):
                unroll = 4

                def pairs(i4, res):
                    accs = []
                    for u in range(unroll):
                        pr = g * nlan + i4 * unroll + u
                        acc = (srows[pr, pl.ds(0, nlan)]
                               * drows[pr, pl.ds(0, nlan)])
                        for j in range(1, out // nlan):
                            acc = acc + (srows[pr, pl.ds(j * nlan, nlan)]
                                         * drows[pr, pl.ds(j * nlan, nlan)])
                        for perm in perms:
                            acc = acc + _lane_take(acc, perm)
                        accs.append(acc)
                    for u in range(unroll):
                        res = jnp.where(lanes == i4 * unroll + u,
                                        accs[u], res)
                    return res

                res = lax.fori_loop(0, nlan // unroll, pairs,
                                    jnp.zeros((nlan,), jnp.float32))
                ovec[pl.ds(g * nlan, nlan)] = res
                return carry

            lax.fori_loop(0, chunk // nlan, group, 0)
            pltpu.sync_copy(ovec, o_hbm.at[pl.ds(base, chunk)])

    return score


def kernel(x, adj, to_pred, W1, b1, W2, b2, distmult):
    n = x.shape[0]
    out = W2.shape[1]
    p = to_pred.shape[0]

    e = _gcn_embeds(x, adj, W1, b1, W2, b2, distmult)

    info = plsc.get_sparse_core_info()
    nw = info.num_cores * info.num_subcores
    unit = nw * _CHUNK
    p_pad = ((p + unit - 1) // unit) * unit

    src = to_pred[:, 0]
    dst = to_pred[:, 1]
    if p_pad != p:
        zpad = jnp.zeros((p_pad - p,), jnp.int32)
        src = jnp.concatenate([src, zpad])
        dst = jnp.concatenate([dst, zpad])

    scores = _make_score_kernel(n, out, p_pad)(e, src, dst)
    return scores[:p]


# speedup vs baseline: 1.2472x; 1.0264x over previous
import functools

import jax
import jax.numpy as jnp
from jax import lax
from jax.experimental import pallas as pl
from jax.experimental.pallas import tpu as pltpu
from jax.experimental.pallas import tpu_sc as plsc


def _s1_body(x_ref, w1_ref, o_ref):
    o_ref[...] = jnp.dot(x_ref[...], w1_ref[...],
                         preferred_element_type=jnp.float32)


def _layer1_body(adj_ref, s1_ref, b1_ref, w2_ref, t_ref):
    h = jnp.dot(adj_ref[...], s1_ref[...], preferred_element_type=jnp.float32)
    h = jnp.maximum(h + b1_ref[...], 0.0)
    t_ref[...] = jnp.dot(h, w2_ref[...], preferred_element_type=jnp.float32)


def _layer2_body(adj_ref, t_ref, b2_ref, dm_ref, e_ref):
    e = jnp.dot(adj_ref[...], t_ref[...], preferred_element_type=jnp.float32)
    e_ref[...] = (e + b2_ref[...]) * jnp.sqrt(dm_ref[...])


def _pick_bm(n):
    for bm in (400, 256, 200, 128, 80, 40, 16, 8):
        if n % bm == 0:
            return bm
    return n


def _gcn_embeds(x, adj, w1, b1, w2, b2, distmult):
    n, feat = x.shape
    hid = w1.shape[1]
    out = w2.shape[1]
    bm = _pick_bm(n)

    s1 = pl.pallas_call(
        _s1_body,
        out_shape=jax.ShapeDtypeStruct((n, hid), jnp.float32),
        grid=(n // bm,),
        in_specs=[pl.BlockSpec((bm, feat), lambda i: (i, 0)),
                  pl.BlockSpec((feat, hid), lambda i: (0, 0))],
        out_specs=pl.BlockSpec((bm, hid), lambda i: (i, 0)),
    )(x, w1)

    t = pl.pallas_call(
        _layer1_body,
        out_shape=jax.ShapeDtypeStruct((n, out), jnp.float32),
        grid=(n // bm,),
        in_specs=[pl.BlockSpec((bm, n), lambda i: (i, 0)),
                  pl.BlockSpec((n, hid), lambda i: (0, 0)),
                  pl.BlockSpec((hid,), lambda i: (0,)),
                  pl.BlockSpec((hid, out), lambda i: (0, 0))],
        out_specs=pl.BlockSpec((bm, out), lambda i: (i, 0)),
    )(adj, s1, b1, w2)

    e = pl.pallas_call(
        _layer2_body,
        out_shape=jax.ShapeDtypeStruct((n, out), jnp.float32),
        grid=(n // bm,),
        in_specs=[pl.BlockSpec((bm, n), lambda i: (i, 0)),
                  pl.BlockSpec((n, out), lambda i: (0, 0)),
                  pl.BlockSpec((out,), lambda i: (0,)),
                  pl.BlockSpec((out,), lambda i: (0,))],
        out_specs=pl.BlockSpec((bm, out), lambda i: (i, 0)),
    )(adj, t, b2, distmult)

    return e


_CHUNK = 400


def _lane_take(x, idx):
    dnums = lax.GatherDimensionNumbers(
        offset_dims=(), collapsed_slice_dims=(0,), start_index_map=(0,))
    return lax.gather(x, idx[:, None], dnums, slice_sizes=(1,),
                      mode=lax.GatherScatterMode.PROMISE_IN_BOUNDS)


@functools.lru_cache(maxsize=None)
def _make_score_kernel(n, out, p_pad):
    info = plsc.get_sparse_core_info()
    nw = info.num_cores * info.num_subcores
    per_w = p_pad // nw
    chunk = _CHUNK
    n_chunks = per_w // chunk
    nlan = info.num_lanes

    mesh = plsc.VectorSubcoreMesh(core_axis_name="c", subcore_axis_name="s")

    @functools.partial(
        pl.kernel,
        out_type=jax.ShapeDtypeStruct((p_pad,), jnp.float32),
        mesh=mesh,
        scratch_types=[
            pltpu.VMEM((chunk,), jnp.int32),
            pltpu.VMEM((chunk,), jnp.int32),
            pltpu.VMEM((chunk, out), jnp.float32),
            pltpu.VMEM((chunk, out), jnp.float32),
            pltpu.VMEM((chunk,), jnp.float32),
            pltpu.SemaphoreType.DMA,
            pltpu.SemaphoreType.DMA,
        ],
    )
    def score(table, src, dst, o_hbm, sidx, didx, srows, drows, ovec,
              sem_s, sem_d):
        wid = lax.axis_index("s") * info.num_cores + lax.axis_index("c")
        base_w = wid * per_w
        lanes = lax.iota(jnp.int32, nlan)
        perms = [lanes ^ sh for sh in (8, 4, 2, 1)]
        for ci in range(n_chunks):
            base = base_w + ci * chunk
            pltpu.sync_copy(src.at[pl.ds(base, chunk)], sidx)
            pltpu.sync_copy(dst.at[pl.ds(base, chunk)], didx)
            cps = pltpu.async_copy(table.at[sidx], srows, sem_s)
            cpd = pltpu.async_copy(table.at[didx], drows, sem_d)
            cps.wait()
            cpd.wait()

            def group(g, carry):
                unroll = 4

                def pairs(i4, res):
                    accs = []
                    for u in range(unroll):
                        pr = g * nlan + i4 * unroll + u
                        acc = (srows[pr, pl.ds(0, nlan)]
                               * drows[pr, pl.ds(0, nlan)])
                        for j in range(1, out // nlan):
                            acc = acc + (srows[pr, pl.ds(j * nlan, nlan)]
                                         * drows[pr, pl.ds(j * nlan, nlan)])
                        for perm in perms:
                            acc = acc + _lane_take(acc, perm)
                        accs.append(acc)
                    for u in range(unroll):
                        res = jnp.where(lanes == i4 * unroll + u,
                                        accs[u], res)
                    return res

                res = lax.fori_loop(0, nlan // unroll, pairs,
                                    jnp.zeros((nlan,), jnp.float32))
                ovec[pl.ds(g * nlan, nlan)] = res
                return carry

            lax.fori_loop(0, chunk // nlan, group, 0)
            pltpu.sync_copy(ovec, o_hbm.at[pl.ds(base, chunk)])

    return score


def kernel(x, adj, to_pred, W1, b1, W2, b2, distmult):
    n = x.shape[0]
    out = W2.shape[1]
    p = to_pred.shape[0]

    e = _gcn_embeds(x, adj, W1, b1, W2, b2, distmult)

    info = plsc.get_sparse_core_info()
    nw = info.num_cores * info.num_subcores
    unit = nw * _CHUNK
    p_pad = ((p + unit - 1) // unit) * unit

    src = to_pred[:, 0]
    dst = to_pred[:, 1]
    if p_pad != p:
        zpad = jnp.zeros((p_pad - p,), jnp.int32)
        src = jnp.concatenate([src, zpad])
        dst = jnp.concatenate([dst, zpad])

    scores = _make_score_kernel(n, out, p_pad)(e, src, dst)
    return scores[:p]
